# baseline (speedup 1.0000x reference)
import functools

import jax
import jax.numpy as jnp
from jax import lax
from jax.experimental import pallas as pl
from jax.experimental.pallas import tpu as pltpu
from jax.experimental.pallas import tpu_sc as plsc

_CTX = 512
_NCLS = 4
_SEQ = 77
_PRE = 5
_SUF = _SEQ - _PRE - _NCLS
_NCONST = _PRE + _SUF
_REP = 8
_NSLOT = 4


def _make_fill_const(B):
    info = plsc.get_sparse_core_info()
    nc, ns = info.num_cores, info.num_subcores
    nw = nc * ns
    nchunk = B // _REP
    upw = _NCONST * nchunk // nw
    assert _NCONST * nchunk == upw * nw
    mesh = plsc.VectorSubcoreMesh(core_axis_name="c", subcore_axis_name="s")

    @functools.partial(
        pl.kernel,
        out_type=jax.ShapeDtypeStruct((_SEQ, B, _CTX), jnp.float32),
        mesh=mesh,
        scratch_types=[
            pltpu.VMEM((_NSLOT, _REP, _CTX), jnp.float32),
            pltpu.SemaphoreType.DMA,
            pltpu.SemaphoreType.DMA,
        ],
    )
    def fill_const(rep_hbm, out_hbm, buf_v, sem_in, sem_out):
        wid = lax.axis_index("s") * nc + lax.axis_index("c")
        u0 = wid * upw
        row0 = u0 // nchunk
        loads = []
        for i in range(_NSLOT):
            k = jnp.minimum(row0 + i, _NCONST - 1)
            cp = pltpu.make_async_copy(rep_hbm.at[k], buf_v.at[i], sem_in)
            cp.start()
            loads.append(cp)
        for cp in loads:
            cp.wait()
        tail = []
        for j in range(upw):
            u = u0 + j
            k = u // nchunk
            c = u % nchunk
            slot = k - row0
            r = jnp.where(k < _PRE, k, k + _NCLS)
            off = pl.multiple_of(c * _REP, _REP)
            cp = pltpu.make_async_copy(
                buf_v.at[slot], out_hbm.at[r, pl.ds(off, _REP)], sem_out)
            cp.start()
            tail.append(cp)
        for cp in tail:
            cp.wait()

    return fill_const


def _cls_body(const_ref, img_ref, w1_ref, b1_ref, w2_ref, b2_ref, out_ref,
              h_ref):
    del const_ref
    j = pl.program_id(1)

    @pl.when(j == 0)
    def _():
        h_ref[...] = jnp.maximum(
            jnp.dot(img_ref[...], w1_ref[...],
                    preferred_element_type=jnp.float32) + b1_ref[...],
            0.0,
        )

    out_ref[0] = (
        jnp.dot(h_ref[...], w2_ref[...], preferred_element_type=jnp.float32)
        + b2_ref[0]
    )


def _fill_cls(const_filled, img, W1, b1, W2, b2):
    B, F = img.shape
    H = W1.shape[1]
    BB = 1024
    grid = (B // BB, _NCLS)
    return pl.pallas_call(
        _cls_body,
        grid=grid,
        in_specs=[
            pl.BlockSpec(memory_space=pltpu.MemorySpace.HBM),
            pl.BlockSpec((BB, F), lambda i, j: (i, 0)),
            pl.BlockSpec((F, H), lambda i, j: (0, 0)),
            pl.BlockSpec((1, H), lambda i, j: (0, 0)),
            pl.BlockSpec((H, _CTX), lambda i, j: (0, j)),
            pl.BlockSpec((1, 1, _CTX), lambda i, j: (j, 0, 0)),
        ],
        out_specs=pl.BlockSpec((1, BB, _CTX), lambda i, j: (_PRE + j, i, 0)),
        out_shape=jax.ShapeDtypeStruct((_SEQ, B, _CTX), jnp.float32),
        input_output_aliases={0: 0},
        scratch_shapes=[pltpu.VMEM((BB, H), jnp.float32)],
    )(const_filled, img, W1, b1.reshape(1, H), W2,
      b2.reshape(_NCLS, 1, _CTX))


def kernel(img, W1, b1, W2, b2, token_prefix, token_suffix):
    B = img.shape[0]
    pre = token_prefix.reshape(_PRE, _CTX)
    suf = token_suffix.reshape(_SUF, _CTX)
    const_rows = jnp.concatenate([pre, suf], axis=0)
    rep = jnp.broadcast_to(const_rows[:, None, :], (_NCONST, _REP, _CTX))
    const_filled = _make_fill_const(B)(rep)
    out_t = _fill_cls(const_filled, img, W1, b1, W2, b2)
    return jnp.transpose(out_t, (1, 0, 2))


# speedup vs baseline: 1.0407x; 1.0407x over previous
import functools

import jax
import jax.numpy as jnp
from jax import lax
from jax.experimental import pallas as pl
from jax.experimental.pallas import tpu as pltpu
from jax.experimental.pallas import tpu_sc as plsc

_CTX = 512
_NCLS = 4
_SEQ = 77
_PRE = 5
_SUF = _SEQ - _PRE - _NCLS
_NCONST = _PRE + _SUF
_REP = 32
_NSLOT = 4


def _make_fill_const(B):
    info = plsc.get_sparse_core_info()
    nc, ns = info.num_cores, info.num_subcores
    nw = nc * ns
    nchunk = B // _REP
    upw = _NCONST * nchunk // nw
    assert _NCONST * nchunk == upw * nw
    mesh = plsc.VectorSubcoreMesh(core_axis_name="c", subcore_axis_name="s")

    @functools.partial(
        pl.kernel,
        out_type=jax.ShapeDtypeStruct((_SEQ, B, _CTX), jnp.float32),
        mesh=mesh,
        scratch_types=[
            pltpu.VMEM((_NSLOT, _REP, _CTX), jnp.float32),
            pltpu.SemaphoreType.DMA,
            pltpu.SemaphoreType.DMA,
        ],
    )
    def fill_const(rep_hbm, out_hbm, buf_v, sem_in, sem_out):
        wid = lax.axis_index("s") * nc + lax.axis_index("c")
        u0 = wid * upw
        row0 = u0 // nchunk
        loads = []
        for i in range(_NSLOT):
            k = jnp.minimum(row0 + i, _NCONST - 1)
            cp = pltpu.make_async_copy(rep_hbm.at[k], buf_v.at[i], sem_in)
            cp.start()
            loads.append(cp)
        for cp in loads:
            cp.wait()
        tail = []
        for j in range(upw):
            u = u0 + j
            k = u // nchunk
            c = u % nchunk
            slot = k - row0
            r = jnp.where(k < _PRE, k, k + _NCLS)
            off = pl.multiple_of(c * _REP, _REP)
            cp = pltpu.make_async_copy(
                buf_v.at[slot], out_hbm.at[r, pl.ds(off, _REP)], sem_out)
            cp.start()
            tail.append(cp)
        for cp in tail:
            cp.wait()

    return fill_const


def _cls_body(const_ref, img_ref, w1_ref, b1_ref, w2_ref, b2_ref, out_ref,
              h_ref):
    del const_ref
    j = pl.program_id(1)

    @pl.when(j == 0)
    def _():
        h_ref[...] = jnp.maximum(
            jnp.dot(img_ref[...], w1_ref[...],
                    preferred_element_type=jnp.float32) + b1_ref[...],
            0.0,
        )

    out_ref[0] = (
        jnp.dot(h_ref[...], w2_ref[...], preferred_element_type=jnp.float32)
        + b2_ref[0]
    )


def _fill_cls(const_filled, img, W1, b1, W2, b2):
    B, F = img.shape
    H = W1.shape[1]
    BB = 1024
    grid = (B // BB, _NCLS)
    return pl.pallas_call(
        _cls_body,
        grid=grid,
        in_specs=[
            pl.BlockSpec(memory_space=pltpu.MemorySpace.HBM),
            pl.BlockSpec((BB, F), lambda i, j: (i, 0)),
            pl.BlockSpec((F, H), lambda i, j: (0, 0)),
            pl.BlockSpec((1, H), lambda i, j: (0, 0)),
            pl.BlockSpec((H, _CTX), lambda i, j: (0, j)),
            pl.BlockSpec((1, 1, _CTX), lambda i, j: (j, 0, 0)),
        ],
        out_specs=pl.BlockSpec((1, BB, _CTX), lambda i, j: (_PRE + j, i, 0)),
        out_shape=jax.ShapeDtypeStruct((_SEQ, B, _CTX), jnp.float32),
        input_output_aliases={0: 0},
        scratch_shapes=[pltpu.VMEM((BB, H), jnp.float32)],
    )(const_filled, img, W1, b1.reshape(1, H), W2,
      b2.reshape(_NCLS, 1, _CTX))


def kernel(img, W1, b1, W2, b2, token_prefix, token_suffix):
    B = img.shape[0]
    pre = token_prefix.reshape(_PRE, _CTX)
    suf = token_suffix.reshape(_SUF, _CTX)
    const_rows = jnp.concatenate([pre, suf], axis=0)
    rep = jnp.broadcast_to(const_rows[:, None, :], (_NCONST, _REP, _CTX))
    const_filled = _make_fill_const(B)(rep)
    out_t = _fill_cls(const_filled, img, W1, b1, W2, b2)
    return jnp.transpose(out_t, (1, 0, 2))
